# Initial kernel scaffold; baseline (speedup 1.0000x reference)
import jax
import jax.numpy as jnp
from jax.experimental import pallas as pl


def kernel(x, expert_idx, w1, v1, w2):
    raise NotImplementedError("write your pallas kernel here")


# speedup vs baseline: 1.2926x; 1.2926x over previous
import jax
import jax.numpy as jnp
from jax.experimental import pallas as pl
from jax.experimental.pallas import tpu as pltpu

E = 8
FFN = 4096
H = 1024
T = 512
BF = 512
NBF = FFN // BF


def _glu_body(eidx_ref, x_ref, w1_ref, v1_ref, w2_ref, o_ref):
    f = pl.program_id(0)
    x = x_ref[...]
    h1 = jax.lax.dot_general(
        x, w1_ref[...], (((1,), (1,)), ((), ())),
        preferred_element_type=jnp.float32)
    h2 = jax.lax.dot_general(
        x, v1_ref[...], (((1,), (1,)), ((), ())),
        preferred_element_type=jnp.float32)
    g = h1 * jax.lax.logistic(h1) * h2
    contrib = jnp.dot(g, w2_ref[...], preferred_element_type=jnp.float32)

    @pl.when(f == 0)
    def _():
        o_ref[...] = contrib

    @pl.when(f != 0)
    def _():
        o_ref[...] = o_ref[...] + contrib


def kernel(x, expert_idx, w1, v1, w2):
    eidx = jnp.asarray(expert_idx, dtype=jnp.int32).reshape((1,))

    def _w_map(f, e):
        return (e[0] * NBF + f, 0)

    grid_spec = pltpu.PrefetchScalarGridSpec(
        num_scalar_prefetch=1,
        grid=(NBF,),
        in_specs=[
            pl.BlockSpec((T, H), lambda f, e: (0, 0)),
            pl.BlockSpec((BF, H), _w_map),
            pl.BlockSpec((BF, H), _w_map),
            pl.BlockSpec((BF, H), _w_map),
        ],
        out_specs=pl.BlockSpec((T, H), lambda f, e: (0, 0)),
    )

    return pl.pallas_call(
        _glu_body,
        grid_spec=grid_spec,
        out_shape=jax.ShapeDtypeStruct((T, H), jnp.float32),
        compiler_params=pltpu.CompilerParams(
            dimension_semantics=("arbitrary",)),
    )(eidx, x, w1, v1, w2)
